# Initial kernel scaffold; baseline (speedup 1.0000x reference)
import jax
import jax.numpy as jnp
from jax.experimental import pallas as pl


def kernel(x, edge_index, edge_attr, batch, Wn1, bn1, Wn2, bn2, We1, be1, We2, be2, W0a, b0a, W0b, b0b, W1a, b1a, W1b, b1b, W2a, b2a, W2b, b2b):
    raise NotImplementedError("write your pallas kernel here")


# speedup vs baseline: 3.5008x; 3.5008x over previous
import functools

import jax
import jax.numpy as jnp
from jax import lax
from jax.experimental import pallas as pl
from jax.experimental.pallas import tpu as pltpu
from jax.experimental.pallas import tpu_sc as plsc

N = 10000
E = 320000
DN = 128
DE = 16
H = 64
OUT = 64
G = 64
EPS = 0.1

NC = 2
NS = 16
NW = NC * NS
ET = E // NW
C = 80
NCH = ET // C
NP = 10240
NROWS = NP // NS
ZR = 128
VL = 16


def _mlp_body(x_ref, w1_ref, b1_ref, w2_ref, b2_ref, o_ref):
    t = jnp.dot(x_ref[...], w1_ref[...], preferred_element_type=jnp.float32)
    t = jnp.maximum(t + b1_ref[...], 0.0)
    o_ref[...] = jnp.dot(t, w2_ref[...], preferred_element_type=jnp.float32) + b2_ref[...]


def _mlp2_tc(x, W1, b1, W2, b2, block_rows):
    R, FI = x.shape
    HH = W1.shape[1]
    FO = W2.shape[1]
    return pl.pallas_call(
        _mlp_body,
        grid=(R // block_rows,),
        in_specs=[
            pl.BlockSpec((block_rows, FI), lambda i: (i, 0)),
            pl.BlockSpec((FI, HH), lambda i: (0, 0)),
            pl.BlockSpec((1, HH), lambda i: (0, 0)),
            pl.BlockSpec((HH, FO), lambda i: (0, 0)),
            pl.BlockSpec((1, FO), lambda i: (0, 0)),
        ],
        out_specs=pl.BlockSpec((block_rows, FO), lambda i: (i, 0)),
        out_shape=jax.ShapeDtypeStruct((R, FO), jnp.float32),
    )(x, W1, b1.reshape(1, -1), W2, b2.reshape(1, -1))


def _combine_body(p_ref, h_ref, wa_ref, ba_ref, wb_ref, bb_ref, o_ref):
    hcur = h_ref[...]
    t = p_ref[0] + p_ref[1] + (1.0 + EPS) * hcur
    u = jnp.dot(t, wa_ref[...], preferred_element_type=jnp.float32)
    u = jnp.maximum(u + ba_ref[...], 0.0)
    u = jnp.dot(u, wb_ref[...], preferred_element_type=jnp.float32) + bb_ref[...]
    o_ref[...] = jnp.maximum(hcur + u, 0.0)


def _combine_tc(part, h, Wa, ba, Wb, bb, block_rows=1000):
    return pl.pallas_call(
        _combine_body,
        grid=(N // block_rows,),
        in_specs=[
            pl.BlockSpec((NC, block_rows, H), lambda i: (0, i, 0)),
            pl.BlockSpec((block_rows, H), lambda i: (i, 0)),
            pl.BlockSpec((H, H), lambda i: (0, 0)),
            pl.BlockSpec((1, H), lambda i: (0, 0)),
            pl.BlockSpec((H, H), lambda i: (0, 0)),
            pl.BlockSpec((1, H), lambda i: (0, 0)),
        ],
        out_specs=pl.BlockSpec((block_rows, H), lambda i: (i, 0)),
        out_shape=jax.ShapeDtypeStruct((N, H), jnp.float32),
    )(part, h, Wa, ba.reshape(1, -1), Wb, bb.reshape(1, -1))


def _pool_body(p_ref, h_ref, b_ref, wa_ref, ba_ref, wb_ref, bb_ref, o_ref):
    i = pl.program_id(0)
    t = p_ref[0] + p_ref[1] + (1.0 + EPS) * h_ref[...]
    u = jnp.dot(t, wa_ref[...], preferred_element_type=jnp.float32)
    u = jnp.maximum(u + ba_ref[...], 0.0)
    h2 = jnp.dot(u, wb_ref[...], preferred_element_type=jnp.float32) + bb_ref[...]
    b = b_ref[0, 0, :]
    oh = (lax.broadcasted_iota(jnp.int32, (G, b.shape[0]), 0) == b[None, :])
    acc = jnp.dot(oh.astype(jnp.float32), h2, preferred_element_type=jnp.float32)

    @pl.when(i == 0)
    def _():
        o_ref[...] = acc

    @pl.when(i != 0)
    def _():
        o_ref[...] += acc


def _pool_tc(part, h, batch3, Wa, ba, Wb, bb, block_rows=1000):
    nb = N // block_rows
    return pl.pallas_call(
        _pool_body,
        grid=(nb,),
        in_specs=[
            pl.BlockSpec((NC, block_rows, H), lambda i: (0, i, 0)),
            pl.BlockSpec((block_rows, H), lambda i: (i, 0)),
            pl.BlockSpec((1, 1, block_rows), lambda i: (i, 0, 0)),
            pl.BlockSpec((H, H), lambda i: (0, 0)),
            pl.BlockSpec((1, H), lambda i: (0, 0)),
            pl.BlockSpec((H, OUT), lambda i: (0, 0)),
            pl.BlockSpec((1, OUT), lambda i: (0, 0)),
        ],
        out_specs=pl.BlockSpec((G, OUT), lambda i: (0, 0)),
        out_shape=jax.ShapeDtypeStruct((G, OUT), jnp.float32),
    )(part, h, batch3, Wa, ba.reshape(1, -1), Wb, bb.reshape(1, -1))


def _sc_body(h_hbm, e_hbm, src_hbm, dst_hbm, out_hbm,
             agg_sh, src_v, dst_v, rows_v, e_v, m_v, z_v, sem):
    c = lax.axis_index("c")
    s = lax.axis_index("s")
    wid = c * NS + s

    zero16 = jnp.zeros((VL,), jnp.float32)

    @pl.loop(0, ZR)
    def _(r):
        for kk in range(H // VL):
            z_v[r, pl.ds(kk * VL, VL)] = zero16

    @pl.loop(0, NROWS // ZR)
    def _(t):
        pltpu.sync_copy(z_v, agg_sh.at[pl.ds(s * NROWS + t * ZR, ZR)])

    plsc.subcore_barrier()

    pltpu.sync_copy(src_hbm.at[wid], src_v)
    pltpu.sync_copy(dst_hbm.at[wid], dst_v)
    ebase = wid * ET

    @pl.loop(0, NCH)
    def _(j):
        pltpu.async_copy(h_hbm.at[src_v.at[j]], rows_v, sem).wait()
        pltpu.sync_copy(e_hbm.at[pl.ds(ebase + j * C, C)], e_v)

        @pl.loop(0, C)
        def _(r):
            for kk in range(H // VL):
                sl = pl.ds(kk * VL, VL)
                m_v[r, sl] = jnp.maximum(rows_v[r, sl] + e_v[r, sl], 0.0)

        pltpu.sync_copy(m_v, agg_sh.at[dst_v.at[j]], add=True)

    plsc.subcore_barrier()

    @pl.loop(0, NROWS // ZR)
    def _(t):
        base = s * NROWS + t * ZR
        pltpu.sync_copy(agg_sh.at[pl.ds(base, ZR)], z_v)
        pltpu.sync_copy(z_v, out_hbm.at[c, pl.ds(base, ZR)])


@functools.partial(
    pl.kernel,
    out_type=jax.ShapeDtypeStruct((NC, NP, H), jnp.float32),
    mesh=plsc.VectorSubcoreMesh(core_axis_name="c", subcore_axis_name="s",
                                num_cores=NC, num_subcores=NS),
    scratch_types=[
        pltpu.VMEM_SHARED((NP, H), jnp.float32),
        pltpu.VMEM((NCH, C), jnp.int32),
        pltpu.VMEM((NCH, C), jnp.int32),
        pltpu.VMEM((C, H), jnp.float32),
        pltpu.VMEM((C, H), jnp.float32),
        pltpu.VMEM((C, H), jnp.float32),
        pltpu.VMEM((ZR, H), jnp.float32),
        pltpu.SemaphoreType.DMA,
    ],
    compiler_params=pltpu.CompilerParams(use_tc_tiling_on_sc=False),
)
def _sc_layer(h_hbm, e_hbm, src_hbm, dst_hbm, out_hbm,
              agg_sh, src_v, dst_v, rows_v, e_v, m_v, z_v, sem):
    _sc_body(h_hbm, e_hbm, src_hbm, dst_hbm, out_hbm,
             agg_sh, src_v, dst_v, rows_v, e_v, m_v, z_v, sem)


def kernel(x, edge_index, edge_attr, batch,
           Wn1, bn1, Wn2, bn2, We1, be1, We2, be2,
           W0a, b0a, W0b, b0b, W1a, b1a, W1b, b1b, W2a, b2a, W2b, b2b):
    src2 = edge_index[0].reshape(NW, NCH, C)
    dst2 = edge_index[1].reshape(NW, NCH, C)
    batch3 = batch.reshape(N // 1000, 1, 1000)

    h = _mlp2_tc(x, Wn1, bn1, Wn2, bn2, 1000)
    e = _mlp2_tc(edge_attr, We1, be1, We2, be2, 2000)

    for (Wa, ba, Wb, bb) in ((W0a, b0a, W0b, b0b), (W1a, b1a, W1b, b1b)):
        part = _sc_layer(h, e, src2, dst2)
        h = _combine_tc(part, h, Wa, ba, Wb, bb)

    part = _sc_layer(h, e, src2, dst2)
    return _pool_tc(part, h, batch3, W2a, b2a, W2b, b2b)


# speedup vs baseline: 3.9105x; 1.1170x over previous
import functools

import jax
import jax.numpy as jnp
from jax import lax
from jax.experimental import pallas as pl
from jax.experimental.pallas import tpu as pltpu
from jax.experimental.pallas import tpu_sc as plsc

N = 10000
E = 320000
DN = 128
DE = 16
H = 64
OUT = 64
G = 64
EPS = 0.1

NC = 2
NS = 16
NW = NC * NS
ET = E // NW
C = 80
NCH = ET // C
NP = 10240
NROWS = NP // NS
ZR = 128
VL = 16


def _mlp_body(x_ref, w1_ref, b1_ref, w2_ref, b2_ref, o_ref):
    t = jnp.dot(x_ref[...], w1_ref[...], preferred_element_type=jnp.float32)
    t = jnp.maximum(t + b1_ref[...], 0.0)
    o_ref[...] = jnp.dot(t, w2_ref[...], preferred_element_type=jnp.float32) + b2_ref[...]


def _mlp2_tc(x, W1, b1, W2, b2, block_rows):
    R, FI = x.shape
    HH = W1.shape[1]
    FO = W2.shape[1]
    return pl.pallas_call(
        _mlp_body,
        grid=(R // block_rows,),
        in_specs=[
            pl.BlockSpec((block_rows, FI), lambda i: (i, 0)),
            pl.BlockSpec((FI, HH), lambda i: (0, 0)),
            pl.BlockSpec((1, HH), lambda i: (0, 0)),
            pl.BlockSpec((HH, FO), lambda i: (0, 0)),
            pl.BlockSpec((1, FO), lambda i: (0, 0)),
        ],
        out_specs=pl.BlockSpec((block_rows, FO), lambda i: (i, 0)),
        out_shape=jax.ShapeDtypeStruct((R, FO), jnp.float32),
    )(x, W1, b1.reshape(1, -1), W2, b2.reshape(1, -1))


def _combine_body(p_ref, h_ref, wa_ref, ba_ref, wb_ref, bb_ref, o_ref):
    hcur = h_ref[...]
    t = p_ref[0] + p_ref[1] + (1.0 + EPS) * hcur
    u = jnp.dot(t, wa_ref[...], preferred_element_type=jnp.float32)
    u = jnp.maximum(u + ba_ref[...], 0.0)
    u = jnp.dot(u, wb_ref[...], preferred_element_type=jnp.float32) + bb_ref[...]
    o_ref[...] = jnp.maximum(hcur + u, 0.0)


def _combine_tc(part, h, Wa, ba, Wb, bb, block_rows=1000):
    return pl.pallas_call(
        _combine_body,
        grid=(N // block_rows,),
        in_specs=[
            pl.BlockSpec((NC, block_rows, H), lambda i: (0, i, 0)),
            pl.BlockSpec((block_rows, H), lambda i: (i, 0)),
            pl.BlockSpec((H, H), lambda i: (0, 0)),
            pl.BlockSpec((1, H), lambda i: (0, 0)),
            pl.BlockSpec((H, H), lambda i: (0, 0)),
            pl.BlockSpec((1, H), lambda i: (0, 0)),
        ],
        out_specs=pl.BlockSpec((block_rows, H), lambda i: (i, 0)),
        out_shape=jax.ShapeDtypeStruct((N, H), jnp.float32),
    )(part, h, Wa, ba.reshape(1, -1), Wb, bb.reshape(1, -1))


def _pool_body(p_ref, h_ref, b_ref, wa_ref, ba_ref, wb_ref, bb_ref, o_ref):
    i = pl.program_id(0)
    t = p_ref[0] + p_ref[1] + (1.0 + EPS) * h_ref[...]
    u = jnp.dot(t, wa_ref[...], preferred_element_type=jnp.float32)
    u = jnp.maximum(u + ba_ref[...], 0.0)
    h2 = jnp.dot(u, wb_ref[...], preferred_element_type=jnp.float32) + bb_ref[...]
    b = b_ref[0, 0, :]
    oh = (lax.broadcasted_iota(jnp.int32, (G, b.shape[0]), 0) == b[None, :])
    acc = jnp.dot(oh.astype(jnp.float32), h2, preferred_element_type=jnp.float32)

    @pl.when(i == 0)
    def _():
        o_ref[...] = acc

    @pl.when(i != 0)
    def _():
        o_ref[...] += acc


def _pool_tc(part, h, batch3, Wa, ba, Wb, bb, block_rows=1000):
    nb = N // block_rows
    return pl.pallas_call(
        _pool_body,
        grid=(nb,),
        in_specs=[
            pl.BlockSpec((NC, block_rows, H), lambda i: (0, i, 0)),
            pl.BlockSpec((block_rows, H), lambda i: (i, 0)),
            pl.BlockSpec((1, 1, block_rows), lambda i: (i, 0, 0)),
            pl.BlockSpec((H, H), lambda i: (0, 0)),
            pl.BlockSpec((1, H), lambda i: (0, 0)),
            pl.BlockSpec((H, OUT), lambda i: (0, 0)),
            pl.BlockSpec((1, OUT), lambda i: (0, 0)),
        ],
        out_specs=pl.BlockSpec((G, OUT), lambda i: (0, 0)),
        out_shape=jax.ShapeDtypeStruct((G, OUT), jnp.float32),
    )(part, h, batch3, Wa, ba.reshape(1, -1), Wb, bb.reshape(1, -1))


def _sc_body(h_hbm, e_hbm, src_hbm, dst_hbm, out_hbm,
             agg_sh, src_v, dst_v, r0, r1, e0, e1, m0, m1, z_v,
             gsem0, gsem1, esem0, esem1, ssem0, ssem1):
    c = lax.axis_index("c")
    s = lax.axis_index("s")
    wid = c * NS + s

    zero16 = jnp.zeros((VL,), jnp.float32)

    @pl.loop(0, ZR, unroll=4)
    def _(r):
        for kk in range(H // VL):
            z_v[r, pl.ds(kk * VL, VL)] = zero16

    @pl.loop(0, NROWS // ZR)
    def _(t):
        pltpu.sync_copy(z_v, agg_sh.at[pl.ds(s * NROWS + t * ZR, ZR)])

    pltpu.async_copy(src_hbm.at[wid], src_v, gsem0)
    pltpu.async_copy(dst_hbm.at[wid], dst_v, esem0)
    pltpu.make_async_copy(src_hbm.at[wid], src_v, gsem0).wait()
    pltpu.make_async_copy(dst_hbm.at[wid], dst_v, esem0).wait()

    plsc.subcore_barrier()

    ebase = wid * ET

    def issue(j, rv, ev, gsem, esem):
        pltpu.async_copy(h_hbm.at[src_v.at[j]], rv, gsem)
        pltpu.async_copy(e_hbm.at[pl.ds(ebase + j * C, C)], ev, esem)

    issue(0, r0, e0, gsem0, esem0)

    @pl.loop(0, NCH)
    def _(j):
        def step(rv, ev, mv, gsem, esem, ssem, rvn, evn, gsemn, esemn):
            @pl.when(j + 1 < NCH)
            def _():
                issue(j + 1, rvn, evn, gsemn, esemn)

            pltpu.make_async_copy(h_hbm.at[src_v.at[j]], rv, gsem).wait()
            pltpu.make_async_copy(
                e_hbm.at[pl.ds(ebase + j * C, C)], ev, esem).wait()

            @pl.when(j >= 2)
            def _():
                pltpu.make_async_copy(
                    mv, agg_sh.at[dst_v.at[j - 2]], ssem).wait()

            @pl.loop(0, C, unroll=4)
            def _(r):
                for kk in range(H // VL):
                    sl = pl.ds(kk * VL, VL)
                    mv[r, sl] = jnp.maximum(rv[r, sl] + ev[r, sl], 0.0)

            pltpu.async_copy(mv, agg_sh.at[dst_v.at[j]], ssem, add=True)

        @pl.when(j % 2 == 0)
        def _():
            step(r0, e0, m0, gsem0, esem0, ssem0, r1, e1, gsem1, esem1)

        @pl.when(j % 2 == 1)
        def _():
            step(r1, e1, m1, gsem1, esem1, ssem1, r0, e0, gsem0, esem0)

    pltpu.make_async_copy(m1, agg_sh.at[dst_v.at[NCH - 2]], ssem1).wait()
    pltpu.make_async_copy(m0, agg_sh.at[dst_v.at[NCH - 1]], ssem0).wait()

    plsc.subcore_barrier()

    @pl.loop(0, NROWS // ZR)
    def _(t):
        base = s * NROWS + t * ZR
        pltpu.sync_copy(agg_sh.at[pl.ds(base, ZR)], z_v)
        pltpu.sync_copy(z_v, out_hbm.at[c, pl.ds(base, ZR)])


@functools.partial(
    pl.kernel,
    out_type=jax.ShapeDtypeStruct((NC, NP, H), jnp.float32),
    mesh=plsc.VectorSubcoreMesh(core_axis_name="c", subcore_axis_name="s",
                                num_cores=NC, num_subcores=NS),
    scratch_types=[
        pltpu.VMEM_SHARED((NP, H), jnp.float32),
        pltpu.VMEM((NCH, C), jnp.int32),
        pltpu.VMEM((NCH, C), jnp.int32),
        pltpu.VMEM((C, H), jnp.float32),
        pltpu.VMEM((C, H), jnp.float32),
        pltpu.VMEM((C, H), jnp.float32),
        pltpu.VMEM((C, H), jnp.float32),
        pltpu.VMEM((C, H), jnp.float32),
        pltpu.VMEM((C, H), jnp.float32),
        pltpu.VMEM((ZR, H), jnp.float32),
        pltpu.SemaphoreType.DMA,
        pltpu.SemaphoreType.DMA,
        pltpu.SemaphoreType.DMA,
        pltpu.SemaphoreType.DMA,
        pltpu.SemaphoreType.DMA,
        pltpu.SemaphoreType.DMA,
    ],
    compiler_params=pltpu.CompilerParams(use_tc_tiling_on_sc=False),
)
def _sc_layer(h_hbm, e_hbm, src_hbm, dst_hbm, out_hbm,
              agg_sh, src_v, dst_v, r0, r1, e0, e1, m0, m1, z_v,
              gsem0, gsem1, esem0, esem1, ssem0, ssem1):
    _sc_body(h_hbm, e_hbm, src_hbm, dst_hbm, out_hbm,
             agg_sh, src_v, dst_v, r0, r1, e0, e1, m0, m1, z_v,
             gsem0, gsem1, esem0, esem1, ssem0, ssem1)


def kernel(x, edge_index, edge_attr, batch,
           Wn1, bn1, Wn2, bn2, We1, be1, We2, be2,
           W0a, b0a, W0b, b0b, W1a, b1a, W1b, b1b, W2a, b2a, W2b, b2b):
    src2 = edge_index[0].reshape(NW, NCH, C)
    dst2 = edge_index[1].reshape(NW, NCH, C)
    batch3 = batch.reshape(N // 1000, 1, 1000)

    h = _mlp2_tc(x, Wn1, bn1, Wn2, bn2, 1000)
    e = _mlp2_tc(edge_attr, We1, be1, We2, be2, 2000)

    for (Wa, ba, Wb, bb) in ((W0a, b0a, W0b, b0b), (W1a, b1a, W1b, b1b)):
        part = _sc_layer(h, e, src2, dst2)
        h = _combine_tc(part, h, Wa, ba, Wb, bb)

    part = _sc_layer(h, e, src2, dst2)
    return _pool_tc(part, h, batch3, W2a, b2a, W2b, b2b)
